# Initial kernel scaffold; baseline (speedup 1.0000x reference)
import jax
import jax.numpy as jnp
from jax import lax
from jax.experimental import pallas as pl
from jax.experimental.pallas import tpu as pltpu
from jax.experimental.pallas import tpu_sc as plsc

N_NODES = 10000
N_EDGES = 320000
D = 128
N_GRAPHS = 64
N_CLASSES = 10

NC = 2
NS = 16
NW = NC * NS

CHUNK = 128
N_CHUNKS = N_EDGES // CHUNK
ROWS_PER_TILE = N_NODES // NS

ROW_BLK = 1000
N_BLKS = N_NODES // ROW_BLK


def _sc_aggregate_body(x_hbm, src_hbm, dst_hbm, zeros_hbm, out_hbm,
                       src_v, dst_v, rows_v, acc, sem):
    cid = lax.axis_index("c")
    sid = lax.axis_index("s")
    wid = sid * NC + cid

    row0 = sid * ROWS_PER_TILE
    pltpu.sync_copy(zeros_hbm.at[pl.ds(row0, ROWS_PER_TILE)],
                    acc.at[pl.ds(row0, ROWS_PER_TILE)])
    plsc.subcore_barrier()

    n_chunks = jnp.where(wid < N_CHUNKS % NW, N_CHUNKS // NW + 1, N_CHUNKS // NW)

    def body(i, carry):
        c = wid + i * NW
        base = c * CHUNK
        pltpu.sync_copy(src_hbm.at[pl.ds(base, CHUNK)], src_v)
        pltpu.sync_copy(dst_hbm.at[pl.ds(base, CHUNK)], dst_v)
        pltpu.async_copy(x_hbm.at[src_v], rows_v, sem).wait()
        pltpu.sync_copy(rows_v, acc.at[dst_v], add=True)
        return carry

    lax.fori_loop(0, n_chunks, body, 0)
    plsc.subcore_barrier()

    pltpu.sync_copy(acc.at[pl.ds(row0, ROWS_PER_TILE)],
                    out_hbm.at[cid, pl.ds(row0, ROWS_PER_TILE)])


@jax.jit
def _sc_aggregate(x, src, dst, zeros):
    mesh = plsc.VectorSubcoreMesh(core_axis_name="c", subcore_axis_name="s")
    return pl.kernel(
        _sc_aggregate_body,
        out_type=jax.ShapeDtypeStruct((NC, N_NODES, D), jnp.float32),
        mesh=mesh,
        scratch_types=[
            pltpu.VMEM((CHUNK,), jnp.int32),
            pltpu.VMEM((CHUNK,), jnp.int32),
            pltpu.VMEM((CHUNK, D), jnp.float32),
            pltpu.VMEM_SHARED((N_NODES, D), jnp.float32),
            pltpu.SemaphoreType.DMA,
        ],
    )(x, src, dst, zeros)


def _mlp_body(s_ref, x_ref, p0_ref, p1_ref, wa_ref, ba_ref, wb_ref, bb_ref,
              o_ref):
    h = s_ref[0, 0] * x_ref[...] + p0_ref[...] + p1_ref[...]
    t = jnp.maximum(
        jnp.dot(h, wa_ref[...], preferred_element_type=jnp.float32)
        + ba_ref[...], 0.0)
    o = jnp.dot(t, wb_ref[...], preferred_element_type=jnp.float32) + bb_ref[...]
    o_ref[...] = jnp.maximum(o, 0.0)


@jax.jit
def _tc_mlp(s, x, p0, p1, wa, ba, wb, bb):
    blk = lambda r, c: pl.BlockSpec((r, c), lambda i: (i, 0))
    fixed = lambda r, c: pl.BlockSpec((r, c), lambda i: (0, 0))
    return pl.pallas_call(
        _mlp_body,
        grid=(N_BLKS,),
        in_specs=[
            pl.BlockSpec(memory_space=pltpu.SMEM),
            blk(ROW_BLK, D), blk(ROW_BLK, D), blk(ROW_BLK, D),
            fixed(D, D), fixed(1, D), fixed(D, D), fixed(1, D),
        ],
        out_specs=blk(ROW_BLK, D),
        out_shape=jax.ShapeDtypeStruct((N_NODES, D), jnp.float32),
    )(s, x, p0, p1, wa, ba, wb, bb)


def _mlp_pool_head_body(s_ref, x_ref, p0_ref, p1_ref, wa_ref, ba_ref, wb_ref,
                        bb_ref, ids_ref, wh1_ref, bh1_ref, wh2_ref, bh2_ref,
                        o_ref, pool_ref, cnt_ref):
    i = pl.program_id(0)

    @pl.when(i == 0)
    def _init():
        pool_ref[...] = jnp.zeros_like(pool_ref)
        cnt_ref[...] = jnp.zeros_like(cnt_ref)

    h = s_ref[0, 0] * x_ref[...] + p0_ref[...] + p1_ref[...]
    t = jnp.maximum(
        jnp.dot(h, wa_ref[...], preferred_element_type=jnp.float32)
        + ba_ref[...], 0.0)
    h2 = jnp.maximum(
        jnp.dot(t, wb_ref[...], preferred_element_type=jnp.float32)
        + bb_ref[...], 0.0)

    ids = ids_ref[0, 0, :]
    gids = lax.broadcasted_iota(jnp.int32, (N_GRAPHS, ROW_BLK), 0)
    onehot = (ids[None, :] == gids).astype(jnp.float32)
    pool_ref[...] += jnp.dot(onehot, h2, preferred_element_type=jnp.float32)
    cnt_ref[...] += jnp.broadcast_to(
        jnp.sum(onehot, axis=1, keepdims=True), (N_GRAPHS, D))

    @pl.when(i == N_BLKS - 1)
    def _head():
        g = pool_ref[...] / jnp.maximum(cnt_ref[...], 1.0)
        th = jnp.maximum(
            jnp.dot(g, wh1_ref[...], preferred_element_type=jnp.float32)
            + bh1_ref[...], 0.0)
        o_ref[...] = (jnp.dot(th, wh2_ref[...],
                              preferred_element_type=jnp.float32)
                      + bh2_ref[...])


@jax.jit
def _tc_mlp_pool_head(s, x, p0, p1, wa, ba, wb, bb, ids3, wh1, bh1, wh2, bh2):
    blk = lambda r, c: pl.BlockSpec((r, c), lambda i: (i, 0))
    fixed = lambda r, c: pl.BlockSpec((r, c), lambda i: (0, 0))
    return pl.pallas_call(
        _mlp_pool_head_body,
        grid=(N_BLKS,),
        in_specs=[
            pl.BlockSpec(memory_space=pltpu.SMEM),
            blk(ROW_BLK, D), blk(ROW_BLK, D), blk(ROW_BLK, D),
            fixed(D, D), fixed(1, D), fixed(D, D), fixed(1, D),
            pl.BlockSpec((1, 1, ROW_BLK), lambda i: (i, 0, 0)),
            fixed(D, D), fixed(1, D), fixed(D, N_CLASSES), fixed(1, N_CLASSES),
        ],
        out_specs=pl.BlockSpec((N_GRAPHS, N_CLASSES), lambda i: (0, 0)),
        out_shape=jax.ShapeDtypeStruct((N_GRAPHS, N_CLASSES), jnp.float32),
        scratch_shapes=[
            pltpu.VMEM((N_GRAPHS, D), jnp.float32),
            pltpu.VMEM((N_GRAPHS, D), jnp.float32),
        ],
    )(s, x, p0, p1, wa, ba, wb, bb, ids3, wh1, bh1, wh2, bh2)


def kernel(x, edge_index, batch, eps1, W1a, b1a, W1b, b1b,
           eps2, W2a, b2a, W2b, b2b, Wh1, bh1, Wh2, bh2):
    src = edge_index[0].astype(jnp.int32)
    dst = edge_index[1].astype(jnp.int32)
    ids3 = batch.astype(jnp.int32).reshape(N_BLKS, 1, ROW_BLK)
    zeros = jnp.zeros((N_NODES, D), jnp.float32)

    s1 = (1.0 + eps1).reshape(1, 1)
    s2 = (1.0 + eps2).reshape(1, 1)

    p = _sc_aggregate(x, src, dst, zeros)
    h1 = _tc_mlp(s1, x, p[0], p[1], W1a, b1a.reshape(1, D), W1b,
                 b1b.reshape(1, D))
    p2 = _sc_aggregate(h1, src, dst, zeros)
    logits = _tc_mlp_pool_head(
        s2, h1, p2[0], p2[1], W2a, b2a.reshape(1, D), W2b, b2b.reshape(1, D),
        ids3, Wh1, bh1.reshape(1, D), Wh2, bh2.reshape(1, N_CLASSES))
    return logits


# speedup vs baseline: 6.2676x; 6.2676x over previous
import jax
import jax.numpy as jnp
from jax import lax
from jax.experimental import pallas as pl
from jax.experimental.pallas import tpu as pltpu
from jax.experimental.pallas import tpu_sc as plsc

N_NODES = 10000
N_EDGES = 320000
D = 128
N_GRAPHS = 64
N_CLASSES = 10

NC = 2
NS = 16
NW = NC * NS

CHUNK = 128
N_CHUNKS = N_EDGES // CHUNK
ROWS_PER_TILE = 624
ROWS_REM = N_NODES - NS * ROWS_PER_TILE

ROW_BLK = 1000
N_BLKS = N_NODES // ROW_BLK


def _sc_aggregate_body(x_hbm, src_hbm, dst_hbm, zeros_hbm, out_hbm,
                       src_v, dst_v, rows_v, acc, sem):
    cid = lax.axis_index("c")
    sid = lax.axis_index("s")
    wid = sid * NC + cid

    row0 = sid * ROWS_PER_TILE
    pltpu.sync_copy(zeros_hbm.at[pl.ds(row0, ROWS_PER_TILE)],
                    acc.at[pl.ds(row0, ROWS_PER_TILE)])

    @pl.when(sid == NS - 1)
    def _zero_rem():
        pltpu.sync_copy(zeros_hbm.at[pl.ds(NS * ROWS_PER_TILE, ROWS_REM)],
                        acc.at[pl.ds(NS * ROWS_PER_TILE, ROWS_REM)])

    plsc.subcore_barrier()

    n_chunks = jnp.where(wid < N_CHUNKS % NW, N_CHUNKS // NW + 1, N_CHUNKS // NW)

    def body(i, carry):
        c = wid + i * NW
        base = c * CHUNK
        pltpu.sync_copy(src_hbm.at[pl.ds(base, CHUNK)], src_v)
        pltpu.sync_copy(dst_hbm.at[pl.ds(base, CHUNK)], dst_v)
        pltpu.async_copy(x_hbm.at[src_v], rows_v, sem).wait()
        pltpu.sync_copy(rows_v, acc.at[dst_v], add=True)
        return carry

    lax.fori_loop(0, n_chunks, body, 0)
    plsc.subcore_barrier()

    pltpu.sync_copy(acc.at[pl.ds(row0, ROWS_PER_TILE)],
                    out_hbm.at[cid, pl.ds(row0, ROWS_PER_TILE)])

    @pl.when(sid == NS - 1)
    def _write_rem():
        pltpu.sync_copy(acc.at[pl.ds(NS * ROWS_PER_TILE, ROWS_REM)],
                        out_hbm.at[cid, pl.ds(NS * ROWS_PER_TILE, ROWS_REM)])


@jax.jit
def _sc_aggregate(x, src, dst, zeros):
    mesh = plsc.VectorSubcoreMesh(core_axis_name="c", subcore_axis_name="s")
    return pl.kernel(
        _sc_aggregate_body,
        out_type=jax.ShapeDtypeStruct((NC, N_NODES, D), jnp.float32),
        mesh=mesh,
        scratch_types=[
            pltpu.VMEM((CHUNK,), jnp.int32),
            pltpu.VMEM((CHUNK,), jnp.int32),
            pltpu.VMEM((CHUNK, D), jnp.float32),
            pltpu.VMEM_SHARED((N_NODES, D), jnp.float32),
            pltpu.SemaphoreType.DMA,
        ],
    )(x, src, dst, zeros)


def _mlp_body(s_ref, x_ref, p0_ref, p1_ref, wa_ref, ba_ref, wb_ref, bb_ref,
              o_ref):
    h = s_ref[0, 0] * x_ref[...] + p0_ref[...] + p1_ref[...]
    t = jnp.maximum(
        jnp.dot(h, wa_ref[...], preferred_element_type=jnp.float32)
        + ba_ref[...], 0.0)
    o = jnp.dot(t, wb_ref[...], preferred_element_type=jnp.float32) + bb_ref[...]
    o_ref[...] = jnp.maximum(o, 0.0)


@jax.jit
def _tc_mlp(s, x, p0, p1, wa, ba, wb, bb):
    blk = lambda r, c: pl.BlockSpec((r, c), lambda i: (i, 0))
    fixed = lambda r, c: pl.BlockSpec((r, c), lambda i: (0, 0))
    return pl.pallas_call(
        _mlp_body,
        grid=(N_BLKS,),
        in_specs=[
            pl.BlockSpec(memory_space=pltpu.SMEM),
            blk(ROW_BLK, D), blk(ROW_BLK, D), blk(ROW_BLK, D),
            fixed(D, D), fixed(1, D), fixed(D, D), fixed(1, D),
        ],
        out_specs=blk(ROW_BLK, D),
        out_shape=jax.ShapeDtypeStruct((N_NODES, D), jnp.float32),
    )(s, x, p0, p1, wa, ba, wb, bb)


def _mlp_pool_head_body(s_ref, x_ref, p0_ref, p1_ref, wa_ref, ba_ref, wb_ref,
                        bb_ref, ids_ref, wh1_ref, bh1_ref, wh2_ref, bh2_ref,
                        o_ref, pool_ref, cnt_ref):
    i = pl.program_id(0)

    @pl.when(i == 0)
    def _init():
        pool_ref[...] = jnp.zeros_like(pool_ref)
        cnt_ref[...] = jnp.zeros_like(cnt_ref)

    h = s_ref[0, 0] * x_ref[...] + p0_ref[...] + p1_ref[...]
    t = jnp.maximum(
        jnp.dot(h, wa_ref[...], preferred_element_type=jnp.float32)
        + ba_ref[...], 0.0)
    h2 = jnp.maximum(
        jnp.dot(t, wb_ref[...], preferred_element_type=jnp.float32)
        + bb_ref[...], 0.0)

    ids = ids_ref[0, 0, :]
    gids = lax.broadcasted_iota(jnp.int32, (N_GRAPHS, ROW_BLK), 0)
    onehot = (ids[None, :] == gids).astype(jnp.float32)
    pool_ref[...] += jnp.dot(onehot, h2, preferred_element_type=jnp.float32)
    cnt_ref[...] += jnp.broadcast_to(
        jnp.sum(onehot, axis=1, keepdims=True), (N_GRAPHS, D))

    @pl.when(i == N_BLKS - 1)
    def _head():
        g = pool_ref[...] / jnp.maximum(cnt_ref[...], 1.0)
        th = jnp.maximum(
            jnp.dot(g, wh1_ref[...], preferred_element_type=jnp.float32)
            + bh1_ref[...], 0.0)
        o_ref[...] = (jnp.dot(th, wh2_ref[...],
                              preferred_element_type=jnp.float32)
                      + bh2_ref[...])


@jax.jit
def _tc_mlp_pool_head(s, x, p0, p1, wa, ba, wb, bb, ids3, wh1, bh1, wh2, bh2):
    blk = lambda r, c: pl.BlockSpec((r, c), lambda i: (i, 0))
    fixed = lambda r, c: pl.BlockSpec((r, c), lambda i: (0, 0))
    return pl.pallas_call(
        _mlp_pool_head_body,
        grid=(N_BLKS,),
        in_specs=[
            pl.BlockSpec(memory_space=pltpu.SMEM),
            blk(ROW_BLK, D), blk(ROW_BLK, D), blk(ROW_BLK, D),
            fixed(D, D), fixed(1, D), fixed(D, D), fixed(1, D),
            pl.BlockSpec((1, 1, ROW_BLK), lambda i: (i, 0, 0)),
            fixed(D, D), fixed(1, D), fixed(D, N_CLASSES), fixed(1, N_CLASSES),
        ],
        out_specs=pl.BlockSpec((N_GRAPHS, N_CLASSES), lambda i: (0, 0)),
        out_shape=jax.ShapeDtypeStruct((N_GRAPHS, N_CLASSES), jnp.float32),
        scratch_shapes=[
            pltpu.VMEM((N_GRAPHS, D), jnp.float32),
            pltpu.VMEM((N_GRAPHS, D), jnp.float32),
        ],
    )(s, x, p0, p1, wa, ba, wb, bb, ids3, wh1, bh1, wh2, bh2)


def kernel(x, edge_index, batch, eps1, W1a, b1a, W1b, b1b,
           eps2, W2a, b2a, W2b, b2b, Wh1, bh1, Wh2, bh2):
    src = edge_index[0].astype(jnp.int32)
    dst = edge_index[1].astype(jnp.int32)
    ids3 = batch.astype(jnp.int32).reshape(N_BLKS, 1, ROW_BLK)
    zeros = jnp.zeros((N_NODES, D), jnp.float32)

    s1 = (1.0 + eps1).reshape(1, 1)
    s2 = (1.0 + eps2).reshape(1, 1)

    p = _sc_aggregate(x, src, dst, zeros)
    h1 = _tc_mlp(s1, x, p[0], p[1], W1a, b1a.reshape(1, D), W1b,
                 b1b.reshape(1, D))
    p2 = _sc_aggregate(h1, src, dst, zeros)
    logits = _tc_mlp_pool_head(
        s2, h1, p2[0], p2[1], W2a, b2a.reshape(1, D), W2b, b2b.reshape(1, D),
        ids3, Wh1, bh1.reshape(1, D), Wh2, bh2.reshape(1, N_CLASSES))
    return logits


# speedup vs baseline: 12.4358x; 1.9841x over previous
import jax
import jax.numpy as jnp
from jax import lax
from jax.experimental import pallas as pl
from jax.experimental.pallas import tpu as pltpu
from jax.experimental.pallas import tpu_sc as plsc

N_NODES = 10000
N_EDGES = 320000
D = 128
N_GRAPHS = 64
N_CLASSES = 10

NC = 2
NS = 16
NW = NC * NS

CHUNK = 128
N_CHUNKS = N_EDGES // CHUNK
ROWS_PER_TILE = 624
ROWS_REM = N_NODES - NS * ROWS_PER_TILE

ROW_BLK = 1000
N_BLKS = N_NODES // ROW_BLK


def _sc_aggregate_body(x_hbm, src_hbm, dst_hbm, zeros_hbm, out_hbm,
                       srci, dsti, rows, acc, gsem, dsem, isem):
    cid = lax.axis_index("c")
    sid = lax.axis_index("s")
    wid = sid * NC + cid

    base_chunks = N_CHUNKS // NW
    rem = N_CHUNKS % NW
    n = jnp.where(wid < rem, base_chunks + 1, base_chunks)
    c0 = wid * base_chunks + jnp.minimum(wid, rem)
    e0 = c0 * CHUNK

    icopy = pltpu.async_copy(src_hbm.at[pl.ds(e0, base_chunks * CHUNK)],
                             srci.at[pl.ds(0, base_chunks * CHUNK)], isem)

    row0 = sid * ROWS_PER_TILE
    pltpu.sync_copy(zeros_hbm.at[pl.ds(row0, ROWS_PER_TILE)],
                    acc.at[pl.ds(row0, ROWS_PER_TILE)])

    @pl.when(sid == NS - 1)
    def _zero_rem():
        pltpu.sync_copy(zeros_hbm.at[pl.ds(NS * ROWS_PER_TILE, ROWS_REM)],
                        acc.at[pl.ds(NS * ROWS_PER_TILE, ROWS_REM)])

    icopy.wait()

    @pl.when(n > base_chunks)
    def _load_extra():
        pltpu.sync_copy(
            src_hbm.at[pl.ds(e0 + base_chunks * CHUNK, CHUNK)],
            srci.at[pl.ds(base_chunks * CHUNK, CHUNK)])

    plsc.subcore_barrier()

    def issue(j):
        slot = lax.rem(j, 2)
        pltpu.async_copy(x_hbm.at[srci.at[pl.ds(j * CHUNK, CHUNK)]],
                         rows.at[slot], gsem.at[slot])
        pltpu.async_copy(dst_hbm.at[pl.ds((c0 + j) * CHUNK, CHUNK)],
                         dsti.at[slot], dsem.at[slot])

    def wait_in(j):
        slot = lax.rem(j, 2)
        pltpu.make_async_copy(x_hbm.at[srci.at[pl.ds(j * CHUNK, CHUNK)]],
                              rows.at[slot], gsem.at[slot]).wait()
        pltpu.make_async_copy(dst_hbm.at[pl.ds((c0 + j) * CHUNK, CHUNK)],
                              dsti.at[slot], dsem.at[slot]).wait()

    issue(0)

    def body(j, carry):
        slot = lax.rem(j, 2)

        @pl.when(j + 1 < n)
        def _prefetch():
            issue(j + 1)

        wait_in(j)
        pltpu.sync_copy(rows.at[slot], acc.at[dsti.at[slot]], add=True)
        return carry

    lax.fori_loop(0, n, body, 0)
    plsc.subcore_barrier()

    pltpu.sync_copy(acc.at[pl.ds(row0, ROWS_PER_TILE)],
                    out_hbm.at[cid, pl.ds(row0, ROWS_PER_TILE)])

    @pl.when(sid == NS - 1)
    def _write_rem():
        pltpu.sync_copy(acc.at[pl.ds(NS * ROWS_PER_TILE, ROWS_REM)],
                        out_hbm.at[cid, pl.ds(NS * ROWS_PER_TILE, ROWS_REM)])


@jax.jit
def _sc_aggregate(x, src, dst, zeros):
    mesh = plsc.VectorSubcoreMesh(core_axis_name="c", subcore_axis_name="s")
    return pl.kernel(
        _sc_aggregate_body,
        out_type=jax.ShapeDtypeStruct((NC, N_NODES, D), jnp.float32),
        mesh=mesh,
        scratch_types=[
            pltpu.VMEM(((N_CHUNKS // NW + 1) * CHUNK,), jnp.int32),
            pltpu.VMEM((2, CHUNK), jnp.int32),
            pltpu.VMEM((2, CHUNK, D), jnp.float32),
            pltpu.VMEM_SHARED((N_NODES, D), jnp.float32),
            pltpu.SemaphoreType.DMA((2,)),
            pltpu.SemaphoreType.DMA((2,)),
            pltpu.SemaphoreType.DMA,
        ],
    )(x, src, dst, zeros)


def _mlp_body(s_ref, x_ref, p0_ref, p1_ref, wa_ref, ba_ref, wb_ref, bb_ref,
              o_ref):
    h = s_ref[0, 0] * x_ref[...] + p0_ref[...] + p1_ref[...]
    t = jnp.maximum(
        jnp.dot(h, wa_ref[...], preferred_element_type=jnp.float32)
        + ba_ref[...], 0.0)
    o = jnp.dot(t, wb_ref[...], preferred_element_type=jnp.float32) + bb_ref[...]
    o_ref[...] = jnp.maximum(o, 0.0)


@jax.jit
def _tc_mlp(s, x, p0, p1, wa, ba, wb, bb):
    blk = lambda r, c: pl.BlockSpec((r, c), lambda i: (i, 0))
    fixed = lambda r, c: pl.BlockSpec((r, c), lambda i: (0, 0))
    return pl.pallas_call(
        _mlp_body,
        grid=(N_BLKS,),
        in_specs=[
            pl.BlockSpec(memory_space=pltpu.SMEM),
            blk(ROW_BLK, D), blk(ROW_BLK, D), blk(ROW_BLK, D),
            fixed(D, D), fixed(1, D), fixed(D, D), fixed(1, D),
        ],
        out_specs=blk(ROW_BLK, D),
        out_shape=jax.ShapeDtypeStruct((N_NODES, D), jnp.float32),
    )(s, x, p0, p1, wa, ba, wb, bb)


def _mlp_pool_head_body(s_ref, x_ref, p0_ref, p1_ref, wa_ref, ba_ref, wb_ref,
                        bb_ref, ids_ref, wh1_ref, bh1_ref, wh2_ref, bh2_ref,
                        o_ref, pool_ref, cnt_ref):
    i = pl.program_id(0)

    @pl.when(i == 0)
    def _init():
        pool_ref[...] = jnp.zeros_like(pool_ref)
        cnt_ref[...] = jnp.zeros_like(cnt_ref)

    h = s_ref[0, 0] * x_ref[...] + p0_ref[...] + p1_ref[...]
    t = jnp.maximum(
        jnp.dot(h, wa_ref[...], preferred_element_type=jnp.float32)
        + ba_ref[...], 0.0)
    h2 = jnp.maximum(
        jnp.dot(t, wb_ref[...], preferred_element_type=jnp.float32)
        + bb_ref[...], 0.0)

    ids = ids_ref[0, 0, :]
    gids = lax.broadcasted_iota(jnp.int32, (N_GRAPHS, ROW_BLK), 0)
    onehot = (ids[None, :] == gids).astype(jnp.float32)
    pool_ref[...] += jnp.dot(onehot, h2, preferred_element_type=jnp.float32)
    cnt_ref[...] += jnp.broadcast_to(
        jnp.sum(onehot, axis=1, keepdims=True), (N_GRAPHS, D))

    @pl.when(i == N_BLKS - 1)
    def _head():
        g = pool_ref[...] / jnp.maximum(cnt_ref[...], 1.0)
        th = jnp.maximum(
            jnp.dot(g, wh1_ref[...], preferred_element_type=jnp.float32)
            + bh1_ref[...], 0.0)
        o_ref[...] = (jnp.dot(th, wh2_ref[...],
                              preferred_element_type=jnp.float32)
                      + bh2_ref[...])


@jax.jit
def _tc_mlp_pool_head(s, x, p0, p1, wa, ba, wb, bb, ids3, wh1, bh1, wh2, bh2):
    blk = lambda r, c: pl.BlockSpec((r, c), lambda i: (i, 0))
    fixed = lambda r, c: pl.BlockSpec((r, c), lambda i: (0, 0))
    return pl.pallas_call(
        _mlp_pool_head_body,
        grid=(N_BLKS,),
        in_specs=[
            pl.BlockSpec(memory_space=pltpu.SMEM),
            blk(ROW_BLK, D), blk(ROW_BLK, D), blk(ROW_BLK, D),
            fixed(D, D), fixed(1, D), fixed(D, D), fixed(1, D),
            pl.BlockSpec((1, 1, ROW_BLK), lambda i: (i, 0, 0)),
            fixed(D, D), fixed(1, D), fixed(D, N_CLASSES), fixed(1, N_CLASSES),
        ],
        out_specs=pl.BlockSpec((N_GRAPHS, N_CLASSES), lambda i: (0, 0)),
        out_shape=jax.ShapeDtypeStruct((N_GRAPHS, N_CLASSES), jnp.float32),
        scratch_shapes=[
            pltpu.VMEM((N_GRAPHS, D), jnp.float32),
            pltpu.VMEM((N_GRAPHS, D), jnp.float32),
        ],
    )(s, x, p0, p1, wa, ba, wb, bb, ids3, wh1, bh1, wh2, bh2)


def kernel(x, edge_index, batch, eps1, W1a, b1a, W1b, b1b,
           eps2, W2a, b2a, W2b, b2b, Wh1, bh1, Wh2, bh2):
    src = edge_index[0].astype(jnp.int32)
    dst = edge_index[1].astype(jnp.int32)
    ids3 = batch.astype(jnp.int32).reshape(N_BLKS, 1, ROW_BLK)
    zeros = jnp.zeros((N_NODES, D), jnp.float32)

    s1 = (1.0 + eps1).reshape(1, 1)
    s2 = (1.0 + eps2).reshape(1, 1)

    p = _sc_aggregate(x, src, dst, zeros)
    h1 = _tc_mlp(s1, x, p[0], p[1], W1a, b1a.reshape(1, D), W1b,
                 b1b.reshape(1, D))
    p2 = _sc_aggregate(h1, src, dst, zeros)
    logits = _tc_mlp_pool_head(
        s2, h1, p2[0], p2[1], W2a, b2a.reshape(1, D), W2b, b2b.reshape(1, D),
        ids3, Wh1, bh1.reshape(1, D), Wh2, bh2.reshape(1, N_CLASSES))
    return logits
